# baseline (speedup 1.0000x reference)
import functools

import jax
import jax.numpy as jnp
from jax import lax
from jax.experimental import pallas as pl
from jax.experimental.pallas import tpu as pltpu
from jax.experimental.pallas import tpu_sc as plsc

_NUM_WORKERS = 32
_CHUNK = 16
_NBUF = 4
_DEPTH = 2


def _make_sc_gather(B, V, D):
    b_per_w = B // _NUM_WORKERS
    n_chunks = b_per_w // _CHUNK
    n_groups = n_chunks // _NBUF
    depth = _DEPTH
    mesh = plsc.VectorSubcoreMesh(core_axis_name="c", subcore_axis_name="s")

    @functools.partial(
        pl.kernel,
        mesh=mesh,
        out_type=jax.ShapeDtypeStruct((B, D), jnp.float32),
        scratch_types=[
            pltpu.VMEM((b_per_w,), jnp.int32),
        ]
        + [pltpu.VMEM((_CHUNK, D), jnp.float32)] * _NBUF
        + [pltpu.SemaphoreType.DMA] * _NBUF
        + [pltpu.SemaphoreType.DMA] * _NBUF,
    )
    def gather_kernel(idx_hbm, table_hbm, out_hbm, idx_v, *bufs_and_sems):
        bufs = bufs_and_sems[:_NBUF]
        gsem = bufs_and_sems[_NBUF:2 * _NBUF]
        osem = bufs_and_sems[2 * _NBUF:]
        wid = lax.axis_index("s") * 2 + lax.axis_index("c")
        base = wid * b_per_w
        pltpu.sync_copy(idx_hbm.at[pl.ds(base, b_per_w)], idx_v)

        def gather(c, j):
            off = pl.multiple_of(c * _CHUNK, 8)
            pltpu.async_copy(
                table_hbm.at[idx_v.at[pl.ds(off, _CHUNK)]], bufs[j], gsem[j])

        def put(c, j):
            off = pl.multiple_of(c * _CHUNK, 8)
            pltpu.async_copy(
                bufs[j], out_hbm.at[pl.ds(base + off, _CHUNK)], osem[j])

        def drain_g(j):
            pltpu.make_async_copy(
                table_hbm.at[pl.ds(0, _CHUNK)], bufs[j], gsem[j]).wait()

        def drain_o(j):
            pltpu.make_async_copy(
                bufs[j], out_hbm.at[pl.ds(base, _CHUNK)], osem[j]).wait()

        for j in range(depth):
            gather(j, j)

        def body(g, carry):
            c0 = g * _NBUF
            for j in range(_NBUF):
                c = c0 + j
                nxt = c + depth
                jn = (j + depth) % _NBUF

                def prefetch(c=c, nxt=nxt, jn=jn):
                    pl.when(c >= _NBUF - depth)(lambda: drain_o(jn))
                    gather(nxt, jn)

                pl.when(nxt < n_chunks)(prefetch)
                drain_g(j)
                put(c, j)
            return carry

        lax.fori_loop(0, n_groups, body, 0)

        for c in range(n_chunks - (_NBUF - depth) - depth, n_chunks):
            drain_o(c % _NBUF)

    return gather_kernel


def kernel(x, pe):
    x_shape = x.shape
    V, D = pe.shape
    flat = x.reshape(-1)
    B = flat.shape[0]
    out = _make_sc_gather(B, V, D)(flat, pe)
    return out.reshape(x_shape + (D,))
